# Initial kernel scaffold; baseline (speedup 1.0000x reference)
import jax
import jax.numpy as jnp
from jax.experimental import pallas as pl


def kernel(feature, condition, edge_index, W_e1, b_e1, W_e2, b_e2, W_e3, b_e3, W_d1, b_d1, W_d2, b_d2, W_d3, b_d3):
    raise NotImplementedError("write your pallas kernel here")


# speedup vs baseline: 8.6707x; 8.6707x over previous
import functools

import jax
import jax.numpy as jnp
from jax import lax
from jax.experimental import pallas as pl
from jax.experimental.pallas import tpu as pltpu
from jax.experimental.pallas import tpu_sc as plsc

_N = 10000
_E = 320000
_NC = 2
_NS = 16
_NW = _NC * _NS
_EPT = _E // _NW
_CH = 80
_NCHUNK = _EPT // _CH
_RB = 624
_RLAST = _N - (_NS - 1) * _RB


def _sc_mesh():
  return plsc.VectorSubcoreMesh(core_axis_name="c", subcore_axis_name="s")


@functools.cache
def _make_scatter_rows(d):
  @functools.partial(
      pl.kernel,
      out_type=jax.ShapeDtypeStruct((_NC, _N, d), jnp.float32),
      mesh=_sc_mesh(),
      scratch_types=[
          pltpu.VMEM((_CH,), jnp.int32),
          pltpu.VMEM((_CH,), jnp.int32),
          pltpu.VMEM((_CH, d), jnp.float32),
          pltpu.VMEM_SHARED((_N, d), jnp.float32),
          pltpu.SemaphoreType.DMA,
      ],
  )
  def scatter_rows(src_hbm, dst_hbm, y_hbm, out_hbm, sidx, didx, rows,
                   acc, sem):
    c = lax.axis_index("c")
    s = lax.axis_index("s")
    wid = c * _NS + s

    def zrow(i, carry):
      for jj in range(d // 16):
        rows[i, pl.ds(jj * 16, 16)] = jnp.zeros((16,), jnp.float32)
      return carry

    lax.fori_loop(0, _CH, zrow, 0)
    row0 = pl.multiple_of(s * _RB, 8)

    @pl.when(s < _NS - 1)
    def _():
      for k in range(_RB // _CH):
        pltpu.sync_copy(rows, acc.at[pl.ds(row0 + k * _CH, _CH)])
      pltpu.sync_copy(rows.at[pl.ds(0, _RB % _CH)],
                      acc.at[pl.ds(row0 + (_RB // _CH) * _CH, _RB % _CH)])

    @pl.when(s == _NS - 1)
    def _():
      for k in range(_RLAST // _CH):
        pltpu.sync_copy(
            rows, acc.at[pl.ds((_NS - 1) * _RB + k * _CH, _CH)])

    plsc.subcore_barrier()

    ebase = wid * _EPT

    def body(j, carry):
      off = pl.multiple_of(ebase + j * _CH, 8)
      pltpu.sync_copy(src_hbm.at[pl.ds(off, _CH)], sidx)
      pltpu.sync_copy(dst_hbm.at[pl.ds(off, _CH)], didx)
      pltpu.async_copy(y_hbm.at[sidx], rows, sem).wait()
      pltpu.sync_copy(rows, acc.at[didx], add=True)
      return carry

    lax.fori_loop(0, _NCHUNK, body, 0)
    plsc.subcore_barrier()

    @pl.when(s < _NS - 1)
    def _():
      pltpu.sync_copy(acc.at[pl.ds(row0, _RB)], out_hbm.at[c, pl.ds(row0, _RB)])

    @pl.when(s == _NS - 1)
    def _():
      pltpu.sync_copy(acc.at[pl.ds((_NS - 1) * _RB, _RLAST)],
                      out_hbm.at[c, pl.ds((_NS - 1) * _RB, _RLAST)])

  return scatter_rows


@functools.cache
def _make_degree():
  @functools.partial(
      pl.kernel,
      out_type=jax.ShapeDtypeStruct((_NC, _N), jnp.float32),
      mesh=_sc_mesh(),
      scratch_types=[
          pltpu.VMEM((_CH,), jnp.int32),
          pltpu.VMEM((_CH,), jnp.float32),
          pltpu.VMEM((_N,), jnp.float32),
          pltpu.VMEM_SHARED((_N,), jnp.float32),
      ],
  )
  def degree(dst_hbm, out_hbm, didx, ones, zbuf, acc):
    c = lax.axis_index("c")
    s = lax.axis_index("s")
    wid = c * _NS + s

    for jj in range(_CH // 16):
      ones[pl.ds(jj * 16, 16)] = jnp.ones((16,), jnp.float32)

    @pl.when(s == 0)
    def _():
      def zrow(i, carry):
        zbuf[pl.ds(i * 16, 16)] = jnp.zeros((16,), jnp.float32)
        return carry

      lax.fori_loop(0, _N // 16, zrow, 0)
      pltpu.sync_copy(zbuf, acc)

    plsc.subcore_barrier()

    ebase = wid * _EPT

    def body(j, carry):
      off = pl.multiple_of(ebase + j * _CH, 8)
      pltpu.sync_copy(dst_hbm.at[pl.ds(off, _CH)], didx)
      pltpu.sync_copy(ones, acc.at[didx], add=True)
      return carry

    lax.fori_loop(0, _NCHUNK, body, 0)
    plsc.subcore_barrier()

    @pl.when(s == 0)
    def _():
      pltpu.sync_copy(acc, out_hbm.at[c])

  return degree


_B = 1000


def _tc_layer1(x1, deg0, deg1, W):
  din, dout = W.shape

  def body(x_ref, d0_ref, d1_ref, w_ref, y_ref, dis_ref):
    dis = lax.rsqrt(d0_ref[...] + d1_ref[...] + 1.0)
    u = jnp.dot(x_ref[...], w_ref[...], preferred_element_type=jnp.float32)
    y_ref[...] = dis * u
    dis_ref[...] = dis

  return pl.pallas_call(
      body,
      grid=(_N // _B,),
      in_specs=[
          pl.BlockSpec((_B, din), lambda i: (i, 0)),
          pl.BlockSpec((_B, 1), lambda i: (i, 0)),
          pl.BlockSpec((_B, 1), lambda i: (i, 0)),
          pl.BlockSpec((din, dout), lambda i: (0, 0)),
      ],
      out_specs=[
          pl.BlockSpec((_B, dout), lambda i: (i, 0)),
          pl.BlockSpec((_B, 1), lambda i: (i, 0)),
      ],
      out_shape=[
          jax.ShapeDtypeStruct((_N, dout), jnp.float32),
          jax.ShapeDtypeStruct((_N, 1), jnp.float32),
      ],
  )(x1, deg0, deg1, W)


def _tc_mid(s0, s1, y, dis, b, W, act_tanh, cond=None, W2=None, pad_to=None):
  din, dout = W.shape
  arr_w = y.shape[1]
  has_cond = cond is not None
  out_w = pad_to if pad_to is not None else dout

  def body(*refs):
    if has_cond:
      s0_ref, s1_ref, y_ref, dis_ref, b_ref, w_ref, c_ref, w2_ref, o_ref = refs
    else:
      s0_ref, s1_ref, y_ref, dis_ref, b_ref, w_ref, o_ref = refs
    dis = dis_ref[...]
    t = (s0_ref[...] + s1_ref[...] + y_ref[...])[:, :din]
    x = dis * t + b_ref[...]
    if act_tanh:
      x = jnp.tanh(x)
    u = jnp.dot(x, w_ref[...], preferred_element_type=jnp.float32)
    if has_cond:
      u = u + jnp.dot(c_ref[...], w2_ref[...],
                      preferred_element_type=jnp.float32)
    u = dis * u
    if pad_to is not None:
      u = jnp.concatenate(
          [u, jnp.zeros((u.shape[0], out_w - dout), jnp.float32)], axis=1)
    o_ref[...] = u

  in_specs = [
      pl.BlockSpec((_B, arr_w), lambda i: (i, 0)),
      pl.BlockSpec((_B, arr_w), lambda i: (i, 0)),
      pl.BlockSpec((_B, arr_w), lambda i: (i, 0)),
      pl.BlockSpec((_B, 1), lambda i: (i, 0)),
      pl.BlockSpec((1, din), lambda i: (0, 0)),
      pl.BlockSpec((din, dout), lambda i: (0, 0)),
  ]
  args = [s0, s1, y, dis, b, W]
  if has_cond:
    in_specs += [
        pl.BlockSpec((_B, cond.shape[1]), lambda i: (i, 0)),
        pl.BlockSpec((cond.shape[1], dout), lambda i: (0, 0)),
    ]
    args += [cond, W2]

  return pl.pallas_call(
      body,
      grid=(_N // _B,),
      in_specs=in_specs,
      out_specs=pl.BlockSpec((_B, out_w), lambda i: (i, 0)),
      out_shape=jax.ShapeDtypeStruct((_N, out_w), jnp.float32),
  )(*args)


def _tc_final(s0, s1, y, dis, b):
  din = y.shape[1]

  def body(s0_ref, s1_ref, y_ref, dis_ref, b_ref, o_ref):
    o_ref[...] = dis_ref[...] * (s0_ref[...] + s1_ref[...] + y_ref[...]) \
        + b_ref[...]

  return pl.pallas_call(
      body,
      grid=(_N // _B,),
      in_specs=[
          pl.BlockSpec((_B, din), lambda i: (i, 0)),
          pl.BlockSpec((_B, din), lambda i: (i, 0)),
          pl.BlockSpec((_B, din), lambda i: (i, 0)),
          pl.BlockSpec((_B, 1), lambda i: (i, 0)),
          pl.BlockSpec((1, din), lambda i: (0, 0)),
      ],
      out_specs=pl.BlockSpec((_B, din), lambda i: (i, 0)),
      out_shape=jax.ShapeDtypeStruct((_N, din), jnp.float32),
  )(s0, s1, y, dis, b)


def kernel(feature, condition, edge_index, W_e1, b_e1, W_e2, b_e2, W_e3, b_e3,
           W_d1, b_d1, W_d2, b_d2, W_d3, b_d3):
  src = edge_index[0]
  dst = edge_index[1]

  deg = _make_degree()(dst)
  deg0 = deg[0].reshape(_N, 1)
  deg1 = deg[1].reshape(_N, 1)

  scat128 = _make_scatter_rows(128)

  x1 = jnp.concatenate([feature, condition], axis=1)
  y1, dis = _tc_layer1(x1, deg0, deg1, W_e1)

  s = scat128(src, dst, y1)
  y2 = _tc_mid(s[0], s[1], y1, dis, b_e1.reshape(1, -1), W_e2, True)
  s = scat128(src, dst, y2)
  y3 = _tc_mid(s[0], s[1], y2, dis, b_e2.reshape(1, -1), W_e3, True,
               pad_to=128)
  s = scat128(src, dst, y3)
  y4 = _tc_mid(s[0], s[1], y3, dis, b_e3.reshape(1, -1), W_d1[:64], False,
               cond=condition, W2=W_d1[64:])
  s = scat128(src, dst, y4)
  y5 = _tc_mid(s[0], s[1], y4, dis, b_d1.reshape(1, -1), W_d2, True)
  s = scat128(src, dst, y5)
  y6 = _tc_mid(s[0], s[1], y5, dis, b_d2.reshape(1, -1), W_d3, True)
  s = scat128(src, dst, y6)
  return _tc_final(s[0], s[1], y6, dis, b_d3.reshape(1, -1))


# speedup vs baseline: 15.4557x; 1.7825x over previous
import functools

import jax
import jax.numpy as jnp
from jax import lax
from jax.experimental import pallas as pl
from jax.experimental.pallas import tpu as pltpu
from jax.experimental.pallas import tpu_sc as plsc

_N = 10000
_E = 320000
_NC = 2
_NS = 16
_NW = _NC * _NS
_EPT = _E // _NW
_CH = 80
_NCHUNK = _EPT // _CH
_RB = 624
_RLAST = _N - (_NS - 1) * _RB


def _sc_mesh():
  return plsc.VectorSubcoreMesh(core_axis_name="c", subcore_axis_name="s")


@functools.cache
def _make_scatter_rows(d):
  @functools.partial(
      pl.kernel,
      out_type=jax.ShapeDtypeStruct((_NC, _N, d), jnp.float32),
      mesh=_sc_mesh(),
      scratch_types=[
          pltpu.VMEM((_EPT,), jnp.int32),
          pltpu.VMEM((_NCHUNK, _CH), jnp.int32),
          pltpu.VMEM((_CH, d), jnp.float32),
          pltpu.VMEM((_CH, d), jnp.float32),
          pltpu.VMEM_SHARED((_N, d), jnp.float32),
          pltpu.SemaphoreType.DMA,
          pltpu.SemaphoreType.DMA,
          pltpu.SemaphoreType.DMA,
          pltpu.SemaphoreType.DMA,
      ],
  )
  def scatter_rows(src_hbm, dst3_hbm, y_hbm, out_hbm, sidx, didx, rows0,
                   rows1, acc, semg0, semg1, sems0, sems1):
    c = lax.axis_index("c")
    s = lax.axis_index("s")
    wid = c * _NS + s

    def zrow(i, carry):
      for jj in range(d // 16):
        rows0[i, pl.ds(jj * 16, 16)] = jnp.zeros((16,), jnp.float32)
      return carry

    lax.fori_loop(0, _CH, zrow, 0)
    row0 = pl.multiple_of(s * _RB, 8)

    @pl.when(s < _NS - 1)
    def _():
      for k in range(_RB // _CH):
        pltpu.sync_copy(rows0, acc.at[pl.ds(row0 + k * _CH, _CH)])
      pltpu.sync_copy(rows0.at[pl.ds(0, _RB % _CH)],
                      acc.at[pl.ds(row0 + (_RB // _CH) * _CH, _RB % _CH)])

    @pl.when(s == _NS - 1)
    def _():
      for k in range(_RLAST // _CH):
        pltpu.sync_copy(
            rows0, acc.at[pl.ds((_NS - 1) * _RB + k * _CH, _CH)])

    ebase = pl.multiple_of(wid * _EPT, 8)
    pltpu.sync_copy(src_hbm.at[pl.ds(ebase, _EPT)], sidx)
    pltpu.sync_copy(dst3_hbm.at[wid], didx)

    plsc.subcore_barrier()

    def gather(j, rb, sg):
      pltpu.async_copy(y_hbm.at[sidx.at[pl.ds(j * _CH, _CH)]], rb, sg)

    def scatter(j, rb, ss):
      pltpu.async_copy(rb, acc.at[didx.at[j]], ss, add=True)

    def wait_dma(dst, sem):
      pltpu.make_async_copy(y_hbm.at[pl.ds(0, _CH)], dst, sem).wait()

    gather(0, rows0, semg0)
    wait_dma(rows0, semg0)
    scatter(0, rows0, sems0)
    gather(1, rows1, semg1)
    wait_dma(rows1, semg1)
    scatter(1, rows1, sems1)
    wait_dma(rows0, sems0)
    gather(2, rows0, semg0)

    def step(i, carry):
      a = 2 * i
      wait_dma(rows0, semg0)
      scatter(a, rows0, sems0)
      wait_dma(rows1, sems1)
      gather(a + 1, rows1, semg1)
      wait_dma(rows1, semg1)
      scatter(a + 1, rows1, sems1)
      wait_dma(rows0, sems0)
      gather(a + 2, rows0, semg0)
      return carry

    lax.fori_loop(1, (_NCHUNK - 1) // 2, step, 0)
    wait_dma(rows0, semg0)
    scatter(_NCHUNK - 1, rows0, sems0)
    wait_dma(rows1, sems1)
    wait_dma(rows0, sems0)
    plsc.subcore_barrier()

    @pl.when(s < _NS - 1)
    def _():
      pltpu.sync_copy(acc.at[pl.ds(row0, _RB)], out_hbm.at[c, pl.ds(row0, _RB)])

    @pl.when(s == _NS - 1)
    def _():
      pltpu.sync_copy(acc.at[pl.ds((_NS - 1) * _RB, _RLAST)],
                      out_hbm.at[c, pl.ds((_NS - 1) * _RB, _RLAST)])

  return scatter_rows


@functools.cache
def _make_degree():
  @functools.partial(
      pl.kernel,
      out_type=jax.ShapeDtypeStruct((_NC, _N), jnp.float32),
      mesh=_sc_mesh(),
      scratch_types=[
          pltpu.VMEM((_CH,), jnp.int32),
          pltpu.VMEM((_CH,), jnp.float32),
          pltpu.VMEM((_N,), jnp.float32),
          pltpu.VMEM_SHARED((_N,), jnp.float32),
      ],
  )
  def degree(dst_hbm, out_hbm, didx, ones, zbuf, acc):
    c = lax.axis_index("c")
    s = lax.axis_index("s")
    wid = c * _NS + s

    for jj in range(_CH // 16):
      ones[pl.ds(jj * 16, 16)] = jnp.ones((16,), jnp.float32)

    @pl.when(s == 0)
    def _():
      def zrow(i, carry):
        zbuf[pl.ds(i * 16, 16)] = jnp.zeros((16,), jnp.float32)
        return carry

      lax.fori_loop(0, _N // 16, zrow, 0)
      pltpu.sync_copy(zbuf, acc)

    plsc.subcore_barrier()

    ebase = wid * _EPT

    def body(j, carry):
      off = pl.multiple_of(ebase + j * _CH, 8)
      pltpu.sync_copy(dst_hbm.at[pl.ds(off, _CH)], didx)
      pltpu.sync_copy(ones, acc.at[didx], add=True)
      return carry

    lax.fori_loop(0, _NCHUNK, body, 0)
    plsc.subcore_barrier()

    @pl.when(s == 0)
    def _():
      pltpu.sync_copy(acc, out_hbm.at[c])

  return degree


_B = 1000


def _tc_layer1(x1, deg0, deg1, W):
  din, dout = W.shape

  def body(x_ref, d0_ref, d1_ref, w_ref, y_ref, dis_ref):
    dis = lax.rsqrt(d0_ref[...] + d1_ref[...] + 1.0)
    u = jnp.dot(x_ref[...], w_ref[...], preferred_element_type=jnp.float32)
    y_ref[...] = dis * u
    dis_ref[...] = dis

  return pl.pallas_call(
      body,
      grid=(_N // _B,),
      in_specs=[
          pl.BlockSpec((_B, din), lambda i: (i, 0)),
          pl.BlockSpec((_B, 1), lambda i: (i, 0)),
          pl.BlockSpec((_B, 1), lambda i: (i, 0)),
          pl.BlockSpec((din, dout), lambda i: (0, 0)),
      ],
      out_specs=[
          pl.BlockSpec((_B, dout), lambda i: (i, 0)),
          pl.BlockSpec((_B, 1), lambda i: (i, 0)),
      ],
      out_shape=[
          jax.ShapeDtypeStruct((_N, dout), jnp.float32),
          jax.ShapeDtypeStruct((_N, 1), jnp.float32),
      ],
  )(x1, deg0, deg1, W)


def _tc_mid(s0, s1, y, dis, b, W, act_tanh, cond=None, W2=None, pad_to=None):
  din, dout = W.shape
  arr_w = y.shape[1]
  has_cond = cond is not None
  out_w = pad_to if pad_to is not None else dout

  def body(*refs):
    if has_cond:
      s0_ref, s1_ref, y_ref, dis_ref, b_ref, w_ref, c_ref, w2_ref, o_ref = refs
    else:
      s0_ref, s1_ref, y_ref, dis_ref, b_ref, w_ref, o_ref = refs
    dis = dis_ref[...]
    t = (s0_ref[...] + s1_ref[...] + y_ref[...])[:, :din]
    x = dis * t + b_ref[...]
    if act_tanh:
      x = jnp.tanh(x)
    u = jnp.dot(x, w_ref[...], preferred_element_type=jnp.float32)
    if has_cond:
      u = u + jnp.dot(c_ref[...], w2_ref[...],
                      preferred_element_type=jnp.float32)
    u = dis * u
    if pad_to is not None:
      u = jnp.concatenate(
          [u, jnp.zeros((u.shape[0], out_w - dout), jnp.float32)], axis=1)
    o_ref[...] = u

  in_specs = [
      pl.BlockSpec((_B, arr_w), lambda i: (i, 0)),
      pl.BlockSpec((_B, arr_w), lambda i: (i, 0)),
      pl.BlockSpec((_B, arr_w), lambda i: (i, 0)),
      pl.BlockSpec((_B, 1), lambda i: (i, 0)),
      pl.BlockSpec((1, din), lambda i: (0, 0)),
      pl.BlockSpec((din, dout), lambda i: (0, 0)),
  ]
  args = [s0, s1, y, dis, b, W]
  if has_cond:
    in_specs += [
        pl.BlockSpec((_B, cond.shape[1]), lambda i: (i, 0)),
        pl.BlockSpec((cond.shape[1], dout), lambda i: (0, 0)),
    ]
    args += [cond, W2]

  return pl.pallas_call(
      body,
      grid=(_N // _B,),
      in_specs=in_specs,
      out_specs=pl.BlockSpec((_B, out_w), lambda i: (i, 0)),
      out_shape=jax.ShapeDtypeStruct((_N, out_w), jnp.float32),
  )(*args)


def _tc_final(s0, s1, y, dis, b):
  din = y.shape[1]

  def body(s0_ref, s1_ref, y_ref, dis_ref, b_ref, o_ref):
    o_ref[...] = dis_ref[...] * (s0_ref[...] + s1_ref[...] + y_ref[...]) \
        + b_ref[...]

  return pl.pallas_call(
      body,
      grid=(_N // _B,),
      in_specs=[
          pl.BlockSpec((_B, din), lambda i: (i, 0)),
          pl.BlockSpec((_B, din), lambda i: (i, 0)),
          pl.BlockSpec((_B, din), lambda i: (i, 0)),
          pl.BlockSpec((_B, 1), lambda i: (i, 0)),
          pl.BlockSpec((1, din), lambda i: (0, 0)),
      ],
      out_specs=pl.BlockSpec((_B, din), lambda i: (i, 0)),
      out_shape=jax.ShapeDtypeStruct((_N, din), jnp.float32),
  )(s0, s1, y, dis, b)


def kernel(feature, condition, edge_index, W_e1, b_e1, W_e2, b_e2, W_e3, b_e3,
           W_d1, b_d1, W_d2, b_d2, W_d3, b_d3):
  src = edge_index[0]
  dst = edge_index[1]
  dst3 = dst.reshape(_NW, _NCHUNK, _CH)

  deg = _make_degree()(dst)
  deg0 = deg[0].reshape(_N, 1)
  deg1 = deg[1].reshape(_N, 1)

  scat128 = _make_scatter_rows(128)

  x1 = jnp.concatenate([feature, condition], axis=1)
  y1, dis = _tc_layer1(x1, deg0, deg1, W_e1)

  s = scat128(src, dst3, y1)
  y2 = _tc_mid(s[0], s[1], y1, dis, b_e1.reshape(1, -1), W_e2, True)
  s = scat128(src, dst3, y2)
  y3 = _tc_mid(s[0], s[1], y2, dis, b_e2.reshape(1, -1), W_e3, True,
               pad_to=128)
  s = scat128(src, dst3, y3)
  y4 = _tc_mid(s[0], s[1], y3, dis, b_e3.reshape(1, -1), W_d1[:64], False,
               cond=condition, W2=W_d1[64:])
  s = scat128(src, dst3, y4)
  y5 = _tc_mid(s[0], s[1], y4, dis, b_d1.reshape(1, -1), W_d2, True)
  s = scat128(src, dst3, y5)
  y6 = _tc_mid(s[0], s[1], y5, dis, b_d2.reshape(1, -1), W_d3, True)
  s = scat128(src, dst3, y6)
  return _tc_final(s[0], s[1], y6, dis, b_d3.reshape(1, -1))
